# baseline (speedup 1.0000x reference)
import functools

import jax
import jax.numpy as jnp
from jax import lax
from jax.experimental import pallas as pl
from jax.experimental.pallas import tpu as pltpu
from jax.experimental.pallas import tpu_sc as plsc

B = 8
S = 384
H = 768
SBLK = 128
NHALF = 2
BH = B // NHALF
NTOKH = BH * S


@functools.cache
def _make_sc_gather(half):
    info = plsc.get_sparse_core_info()
    nc, ns = info.num_cores, info.num_subcores
    nw = nc * ns
    per_w = NTOKH // nw


    mesh = plsc.VectorSubcoreMesh(core_axis_name="c", subcore_axis_name="s")

    @functools.partial(
        pl.kernel,
        mesh=mesh,
        out_type=jax.ShapeDtypeStruct((NTOKH, H), jnp.float32),
        scratch_types=[
            pltpu.VMEM((per_w,), jnp.int32),
            pltpu.VMEM((per_w, H), jnp.float32),
            pltpu.SemaphoreType.DMA,
        ],
    )
    def sc_gather(ids_hbm, table_hbm, out_hbm, idx_v, rows_v, sem):
        wid = lax.axis_index("s") * nc + lax.axis_index("c")
        base = wid * per_w
        pltpu.sync_copy(ids_hbm.at[pl.ds(half * NTOKH + base, per_w)], idx_v)
        pltpu.async_copy(table_hbm.at[idx_v], rows_v, sem).wait()
        pltpu.sync_copy(rows_v, out_hbm.at[pl.ds(base, per_w)])

    return sc_gather


def _ln_body(g_ref, pos_ref, tok_ref, w_ref, b_ref, out_ref):
    x = g_ref[...] + pos_ref[...] + tok_ref[...]
    xt = x.T
    mean = jnp.mean(xt, axis=0, keepdims=True)
    zm = xt - mean
    var = jnp.mean(zm * zm, axis=0, keepdims=True)
    y = zm * lax.rsqrt(var + 1e-5)
    out_ref[0, :, 0, :] = y * w_ref[0, 0][:, None] + b_ref[0, 0][:, None]


def _ln_body_alias(g_ref, pos_ref, tok_ref, w_ref, b_ref, dummy_ref, out_ref):
    del dummy_ref
    _ln_body(g_ref, pos_ref, tok_ref, w_ref, b_ref, out_ref)


def _ln_half_call(g3, pos_emb, tok_emb, w2, b2, half, donated=None, interpret=False):
    off = half * BH
    in_specs = [
        pl.BlockSpec((SBLK, H), lambda sb, i: (i * (S // SBLK) + sb, 0)),
        pl.BlockSpec((SBLK, H), lambda sb, i: (sb, 0)),
        pl.BlockSpec((SBLK, H), lambda sb, i: (sb, 0)),
        pl.BlockSpec((1, 1, H), lambda sb, i: (i + off, 0, 0)),
        pl.BlockSpec((1, 1, H), lambda sb, i: (i + off, 0, 0)),
    ]
    args = [g3, pos_emb, tok_emb, w2, b2]
    kwargs = {}
    body = _ln_body
    if donated is not None:
        in_specs.append(pl.BlockSpec(memory_space=pl.ANY))
        args.append(donated)
        kwargs["input_output_aliases"] = {5: 0}
        body = _ln_body_alias
    return pl.pallas_call(
        body,
        grid=(S // SBLK, BH),
        in_specs=in_specs,
        out_specs=pl.BlockSpec((1, H, 1, SBLK), lambda sb, i: (i + off, 0, 0, sb)),
        out_shape=jax.ShapeDtypeStruct((B, H, 1, S), jnp.float32),
        interpret=interpret,
        **kwargs,
    )(*args)


def kernel(input_ids, word_emb, pos_emb, tok_emb, ln_weight, ln_bias):
    ids = input_ids.reshape(-1).astype(jnp.int32)
    w2 = ln_weight.reshape(B, 1, H)
    b2 = ln_bias.reshape(B, 1, H)
    g0 = _make_sc_gather(0)(ids, word_emb)
    g1 = _make_sc_gather(1)(ids, word_emb)
    out0 = _ln_half_call(g0, pos_emb, tok_emb, w2, b2, half=0)
    return _ln_half_call(g1, pos_emb, tok_emb, w2, b2, half=1, donated=out0)


# speedup vs baseline: 1.2677x; 1.2677x over previous
import functools

import jax
import jax.numpy as jnp
from jax import lax
from jax.experimental import pallas as pl
from jax.experimental.pallas import tpu as pltpu
from jax.experimental.pallas import tpu_sc as plsc

B = 8
S = 384
H = 768
NTOK = B * S


@functools.cache
def _make_sc_gather():
    info = plsc.get_sparse_core_info()
    nc, ns = info.num_cores, info.num_subcores
    nw = nc * ns
    per_w = NTOK // nw

    mesh = plsc.VectorSubcoreMesh(core_axis_name="c", subcore_axis_name="s")

    @functools.partial(
        pl.kernel,
        mesh=mesh,
        out_type=jax.ShapeDtypeStruct((NTOK, H), jnp.float32),
        scratch_types=[
            pltpu.VMEM((per_w,), jnp.int32),
            pltpu.VMEM((per_w, H), jnp.float32),
            pltpu.SemaphoreType.DMA,
        ],
    )
    def sc_gather(ids_hbm, table_hbm, out_hbm, idx_v, rows_v, sem):
        wid = lax.axis_index("s") * nc + lax.axis_index("c")
        base = wid * per_w
        pltpu.sync_copy(ids_hbm.at[pl.ds(base, per_w)], idx_v)
        pltpu.async_copy(table_hbm.at[idx_v], rows_v, sem).wait()
        pltpu.sync_copy(rows_v, out_hbm.at[pl.ds(base, per_w)])

    return sc_gather


def _ln_body(g_ref, pos_ref, tok_ref, w_ref, b_ref, out_ref):
    x = g_ref[...] + pos_ref[...] + tok_ref[...]
    xt = x.T
    mean = jnp.mean(xt, axis=0, keepdims=True)
    zm = xt - mean
    var = jnp.mean(zm * zm, axis=0, keepdims=True)
    y = zm * lax.rsqrt(var + 1e-5)
    out_ref[0, :, 0, :] = y * w_ref[0, 0][:, None] + b_ref[0, 0][:, None]


def _ln_call(garr, pos_emb, tok_emb, w2, b2, interpret=False):
    return pl.pallas_call(
        _ln_body,
        grid=(B,),
        in_specs=[
            pl.BlockSpec((S, H), lambda i: (i, 0)),
            pl.BlockSpec((S, H), lambda i: (0, 0)),
            pl.BlockSpec((S, H), lambda i: (0, 0)),
            pl.BlockSpec((1, 1, H), lambda i: (i, 0, 0)),
            pl.BlockSpec((1, 1, H), lambda i: (i, 0, 0)),
        ],
        out_specs=pl.BlockSpec((1, H, 1, S), lambda i: (i, 0, 0, 0)),
        out_shape=jax.ShapeDtypeStruct((B, H, 1, S), jnp.float32),
        interpret=interpret,
    )(garr, pos_emb, tok_emb, w2, b2)


def kernel(input_ids, word_emb, pos_emb, tok_emb, ln_weight, ln_bias):
    ids = input_ids.reshape(-1).astype(jnp.int32)
    garr = _make_sc_gather()(ids, word_emb)
    return _ln_call(
        garr,
        pos_emb,
        tok_emb,
        ln_weight.reshape(B, 1, H),
        ln_bias.reshape(B, 1, H),
    )
